# baseline (speedup 1.0000x reference)
import functools

import jax
import jax.numpy as jnp
from jax import lax
from jax.experimental import pallas as pl
from jax.experimental.pallas import tpu as pltpu
from jax.experimental.pallas import tpu_sc as plsc

_NUM_CORES = 2
_NUM_SUBCORES = 16
_NUM_WORKERS = _NUM_CORES * _NUM_SUBCORES
_LANE = 16
_BT = 128
_NBUF = 5


@functools.lru_cache(maxsize=None)
def _make_gather(S: int, NB: int, V: int, D: int):
    assert NB == _NUM_WORKERS * _BT and D % 8 == 0 and S % _NBUF == 0
    n_dt = D // 8

    mesh = plsc.VectorSubcoreMesh(core_axis_name="c", subcore_axis_name="s")

    @functools.partial(
        pl.kernel,
        out_type=jax.ShapeDtypeStruct(
            (S, n_dt, _NUM_WORKERS, 8, _BT), jnp.float32
        ),
        mesh=mesh,
        scratch_types=[
            pltpu.VMEM((S, _BT), jnp.int32),
            [pltpu.VMEM((_BT, D), jnp.float32) for _ in range(_NBUF)],
            [pltpu.VMEM((D, _BT + 1), jnp.float32) for _ in range(_NBUF)],
            [pltpu.SemaphoreType.DMA for _ in range(_NBUF)],
            [pltpu.SemaphoreType.DMA for _ in range(_NBUF)],
        ],
        compiler_params=pltpu.CompilerParams(
            use_tc_tiling_on_sc=False, needs_layout_passes=False
        ),
    )
    def gather(idx_hbm, table_hbm, out_hbm, idx_all, rows, trans, gsems, ssems):
        wid = lax.axis_index("s") * _NUM_CORES + lax.axis_index("c")
        pltpu.sync_copy(idx_hbm.at[:, pl.ds(wid * _BT, _BT)], idx_all)

        def gather_desc(s, b):
            return pltpu.make_async_copy(
                table_hbm.at[idx_all.at[s]], rows[b], gsems[b]
            )

        def store_descs(s, b):
            return [
                pltpu.make_async_copy(
                    trans[b].at[pl.ds(dt * 8, 8), pl.ds(0, _BT)],
                    out_hbm.at[s, dt, wid],
                    ssems[b],
                )
                for dt in range(n_dt)
            ]

        lanes = lax.iota(jnp.int32, 16)
        d_ids = [lanes + t * _LANE for t in range(D // _LANE)]
        zeros = lanes * 0

        def transpose(b):
            @pl.loop(0, _BT, unroll=8)
            def _bl(bl):
                bl_vec = zeros + bl
                for t in range(D // _LANE):
                    vals = rows[b][bl, pl.ds(t * _LANE, _LANE)]
                    plsc.store_scatter(trans[b], [d_ids[t], bl_vec], vals)

        for b in range(_NBUF):
            gather_desc(b, b).start()

        @pl.loop(0, S, step=_NBUF)
        def _outer(s0):
            for b in range(_NBUF):
                s = s0 + b
                gather_desc(s, b).wait()

                @pl.when(s >= _NBUF)
                def _wait_prev_store():
                    for d in store_descs(s - _NBUF, b):
                        d.wait()

                transpose(b)

                @pl.when(s + _NBUF < S)
                def _start_next_gather():
                    gather_desc(s + _NBUF, b).start()

                for d in store_descs(s, b):
                    d.start()

        for b in range(_NBUF):
            for d in store_descs(S - _NBUF + b, b):
                d.wait()

    return gather


def kernel(idxs, table):
    b, s = idxs.shape
    v, d = table.shape
    idx_t = idxs.T.astype(jnp.int32)
    out5 = _make_gather(s, b, v, d)(idx_t, table)
    return out5.transpose(2, 4, 0, 1, 3).reshape(b, s, d)


# speedup vs baseline: 1.5422x; 1.5422x over previous
import functools

import jax
import jax.numpy as jnp
from jax import lax
from jax.experimental import pallas as pl
from jax.experimental.pallas import tpu as pltpu
from jax.experimental.pallas import tpu_sc as plsc

_NUM_CORES = 2
_NUM_SUBCORES = 16
_NUM_WORKERS = _NUM_CORES * _NUM_SUBCORES
_LANE = 16
_BT = 128
_NBUF = 5


@functools.lru_cache(maxsize=None)
def _make_gather(S: int, NB: int, V: int, D: int):
    assert NB == _NUM_WORKERS * _BT and D % 8 == 0 and S % _NBUF == 0
    n_dt = D // 8

    mesh = plsc.VectorSubcoreMesh(core_axis_name="c", subcore_axis_name="s")

    @functools.partial(
        pl.kernel,
        out_type=jax.ShapeDtypeStruct(
            (S, n_dt, _NUM_WORKERS, 8, _BT), jnp.float32
        ),
        mesh=mesh,
        scratch_types=[
            pltpu.VMEM((S, _BT), jnp.int32),
            [pltpu.VMEM((_BT, D), jnp.float32) for _ in range(_NBUF)],
            [pltpu.VMEM((D, _BT + 1), jnp.float32) for _ in range(_NBUF)],
            [pltpu.SemaphoreType.DMA for _ in range(_NBUF)],
            [pltpu.SemaphoreType.DMA for _ in range(_NBUF)],
        ],
        compiler_params=pltpu.CompilerParams(
            use_tc_tiling_on_sc=False, needs_layout_passes=False
        ),
    )
    def gather(idx_hbm, table_hbm, out_hbm, idx_all, rows, trans, gsems, ssems):
        wid = lax.axis_index("s") * _NUM_CORES + lax.axis_index("c")
        pltpu.sync_copy(idx_hbm.at[:, pl.ds(wid * _BT, _BT)], idx_all)

        def gather_desc(s, b):
            return pltpu.make_async_copy(
                table_hbm.at[idx_all.at[s]], rows[b], gsems[b]
            )

        def store_descs(s, b):
            return [
                pltpu.make_async_copy(
                    trans[b].at[pl.ds(dt * 8, 8), pl.ds(0, _BT)],
                    out_hbm.at[s, dt, wid],
                    ssems[b],
                )
                for dt in range(n_dt)
            ]

        lanes = lax.iota(jnp.int32, 16)
        d_ids = [lanes + t * _LANE for t in range(D // _LANE)]
        zeros = lanes * 0

        def transpose(b):
            @plsc.parallel_loop(0, _BT, 1, unroll=8)
            def _bl(bl):
                bl_vec = zeros + bl
                for t in range(D // _LANE):
                    vals = rows[b][bl, pl.ds(t * _LANE, _LANE)]
                    plsc.store_scatter(trans[b], [d_ids[t], bl_vec], vals)

        for b in range(_NBUF):
            gather_desc(b, b).start()

        @pl.loop(0, S, step=_NBUF)
        def _outer(s0):
            for b in range(_NBUF):
                s = s0 + b
                gather_desc(s, b).wait()

                @pl.when(s >= _NBUF)
                def _wait_prev_store():
                    for d in store_descs(s - _NBUF, b):
                        d.wait()

                transpose(b)

                @pl.when(s + _NBUF < S)
                def _start_next_gather():
                    gather_desc(s + _NBUF, b).start()

                for d in store_descs(s, b):
                    d.start()

        for b in range(_NBUF):
            for d in store_descs(S - _NBUF + b, b):
                d.wait()

    return gather


def kernel(idxs, table):
    b, s = idxs.shape
    v, d = table.shape
    idx_t = idxs.T.astype(jnp.int32)
    out5 = _make_gather(s, b, v, d)(idx_t, table)
    return out5.transpose(2, 4, 0, 1, 3).reshape(b, s, d)
